# Initial kernel scaffold; baseline (speedup 1.0000x reference)
import jax
import jax.numpy as jnp
from jax.experimental import pallas as pl


def kernel(x, edge_index, edge_attr, batch, Wx, bx, Wg0, bg0, g0, be0, Wg1, bg1, g1, be1, Wg2, bg2, g2, be2, Wf0, bf0, Wf1, bf1, Wf2, bf2):
    raise NotImplementedError("write your pallas kernel here")


# speedup vs baseline: 9.6443x; 9.6443x over previous
import functools

import jax
import jax.numpy as jnp
from jax import lax
from jax.experimental import pallas as pl
from jax.experimental.pallas import tpu as pltpu
from jax.experimental.pallas import tpu_sc as plsc

_N = 10000
_D = 128
_G = 200
_NC = 2
_NS = 16
_NW = _NC * _NS
_K = 128
_NP = 10240
_RPS = _NP // _NS


def _pad_edges(e):
    q = _NW * _K
    return ((e + q - 1) // q) * q


def _deg_body(dst_hbm, out_hbm, didx, ones, zbuf, acc, sem):
    c = lax.axis_index("c")
    s = lax.axis_index("s")
    ept = dst_hbm.shape[0] // _NW
    nchunks = ept // _K

    def _fill(r, _):
        ones[r, :] = jnp.full((16,), 1.0, jnp.float32)
        zbuf[r, :] = jnp.zeros((16,), jnp.float32)
        return 0
    lax.fori_loop(0, _K, _fill, 0)

    base_r = s * _RPS
    for q in range(_RPS // _K):
        pltpu.sync_copy(zbuf, acc.at[pl.ds(base_r + q * _K, _K)])
    plsc.subcore_barrier()

    ebase = (c * _NS + s) * ept

    def _chunk(j, _):
        pltpu.sync_copy(dst_hbm.at[pl.ds(ebase + j * _K, _K)], didx)
        pltpu.sync_copy(ones, acc.at[didx], add=True)
        return 0
    lax.fori_loop(0, nchunks, _chunk, 0)
    plsc.subcore_barrier()

    pltpu.sync_copy(acc.at[pl.ds(base_r, _RPS)],
                    out_hbm.at[c, pl.ds(base_r, _RPS)])


def _make_deg_call(ep):
    mesh = plsc.VectorSubcoreMesh(core_axis_name="c", subcore_axis_name="s")
    return pl.kernel(
        _deg_body,
        out_type=jax.ShapeDtypeStruct((_NC, _NP, 16), jnp.float32),
        mesh=mesh,
        scratch_types=[
            pltpu.VMEM((_K,), jnp.int32),
            pltpu.VMEM((_K, 16), jnp.float32),
            pltpu.VMEM((_K, 16), jnp.float32),
            pltpu.VMEM_SHARED((_NP, 16), jnp.float32),
            pltpu.SemaphoreType.DMA,
        ],
        name="sc_degree_count",
    )


def _scatter_body(hs_hbm, src_hbm, dst_hbm, out_hbm, sidx, didx, rows, acc, sem):
    c = lax.axis_index("c")
    s = lax.axis_index("s")
    ept = src_hbm.shape[0] // _NW
    nchunks = ept // _K

    def _fill(r, _):
        for q in range(8):
            rows[r, pl.ds(q * 16, 16)] = jnp.zeros((16,), jnp.float32)
        return 0
    lax.fori_loop(0, _K, _fill, 0)
    base_r = s * _RPS
    for q in range(_RPS // _K):
        pltpu.sync_copy(rows, acc.at[pl.ds(base_r + q * _K, _K)])
    plsc.subcore_barrier()

    ebase = (c * _NS + s) * ept

    def _chunk(j, _):
        eoff = ebase + j * _K
        pltpu.sync_copy(src_hbm.at[pl.ds(eoff, _K)], sidx)
        pltpu.sync_copy(dst_hbm.at[pl.ds(eoff, _K)], didx)
        pltpu.async_copy(hs_hbm.at[sidx], rows, sem).wait()
        pltpu.sync_copy(rows, acc.at[didx], add=True)
        return 0
    lax.fori_loop(0, nchunks, _chunk, 0)
    plsc.subcore_barrier()

    pltpu.sync_copy(acc.at[pl.ds(base_r, _RPS)],
                    out_hbm.at[c, pl.ds(base_r, _RPS)])


def _make_scatter_call(ep):
    mesh = plsc.VectorSubcoreMesh(core_axis_name="c", subcore_axis_name="s")
    return pl.kernel(
        _scatter_body,
        out_type=jax.ShapeDtypeStruct((_NC, _NP, _D), jnp.float32),
        mesh=mesh,
        scratch_types=[
            pltpu.VMEM((_K,), jnp.int32),
            pltpu.VMEM((_K,), jnp.int32),
            pltpu.VMEM((_K, _D), jnp.float32),
            pltpu.VMEM_SHARED((_NP, _D), jnp.float32),
            pltpu.SemaphoreType.DMA,
        ],
        name="sc_edge_scatter_add",
    )


def _embed_body(x_ref, wx_ref, bx_ref, wg_ref, degs_ref, hs_ref, dinv_ref):
    deg = degs_ref[0, : _N, 0:1] + degs_ref[1, : _N, 0:1] + 1.0
    dinv = lax.rsqrt(deg)
    h0 = jnp.dot(x_ref[...], wx_ref[...],
                 preferred_element_type=jnp.float32) + bx_ref[...]
    hs_ref[...] = jnp.dot(h0, wg_ref[...],
                          preferred_element_type=jnp.float32) * dinv
    dinv_ref[...] = dinv


_embed_call = pl.pallas_call(
    _embed_body,
    out_shape=[
        jax.ShapeDtypeStruct((_N, _D), jnp.float32),
        jax.ShapeDtypeStruct((_N, 1), jnp.float32),
    ],
)


def _mid_body(p_ref, hs_ref, dinv_ref, bg_ref, g_ref, be_ref, wn_ref, out_ref):
    dinv = dinv_ref[...]
    t = dinv * (p_ref[0, : _N, :] + p_ref[1, : _N, :] + hs_ref[...]) + bg_ref[...]
    m = jnp.mean(t, axis=0, keepdims=True)
    d = t - m
    v = jnp.mean(d * d, axis=0, keepdims=True)
    h = d * lax.rsqrt(v + 1e-5) * g_ref[...] + be_ref[...]
    h = jnp.maximum(h, 0.0)
    out_ref[...] = jnp.dot(h, wn_ref[...],
                           preferred_element_type=jnp.float32) * dinv


_mid_call = pl.pallas_call(
    _mid_body,
    out_shape=jax.ShapeDtypeStruct((_N, _D), jnp.float32),
)


def _head_body(p_ref, hs_ref, dinv_ref, bg_ref, g_ref, be_ref, batch_ref,
               wf0_ref, bf0_ref, wf1_ref, bf1_ref, wf2_ref, bf2_ref, out_ref):
    t = (dinv_ref[...] * (p_ref[0, : _N, :] + p_ref[1, : _N, :] + hs_ref[...])
         + bg_ref[...])
    m = jnp.mean(t, axis=0, keepdims=True)
    d = t - m
    v = jnp.mean(d * d, axis=0, keepdims=True)
    h = d * lax.rsqrt(v + 1e-5) * g_ref[...] + be_ref[...]

    gids = lax.broadcasted_iota(jnp.int32, (_G, 1), 0)
    onehot = (batch_ref[...] == gids).astype(jnp.float32)
    sums = jnp.dot(onehot, h, preferred_element_type=jnp.float32)
    cnt = jnp.sum(onehot, axis=1, keepdims=True)
    o = sums / jnp.maximum(cnt, 1.0)

    o = jnp.maximum(jnp.dot(o, wf0_ref[...],
                            preferred_element_type=jnp.float32) + bf0_ref[...], 0.0)
    o = jnp.maximum(jnp.dot(o, wf1_ref[...],
                            preferred_element_type=jnp.float32) + bf1_ref[...], 0.0)
    out_ref[...] = jnp.dot(o, wf2_ref[...],
                           preferred_element_type=jnp.float32) + bf2_ref[...]


_head_call = pl.pallas_call(
    _head_body,
    out_shape=jax.ShapeDtypeStruct((_G, 1), jnp.float32),
)


def kernel(x, edge_index, edge_attr, batch, Wx, bx, Wg0, bg0, g0, be0,
           Wg1, bg1, g1, be1, Wg2, bg2, g2, be2, Wf0, bf0, Wf1, bf1, Wf2, bf2):
    e = edge_index.shape[1]
    ep = _pad_edges(e)
    src = edge_index[0].astype(jnp.int32)
    dst = edge_index[1].astype(jnp.int32)
    pad = ep - e
    src_p = jnp.concatenate([src, jnp.zeros((pad,), jnp.int32)])
    dst_p = jnp.concatenate([dst, jnp.full((pad,), _N, jnp.int32)])

    deg_call = _make_deg_call(ep)
    scatter_call = _make_scatter_call(ep)

    degs = deg_call(dst_p)
    hs0, dinv = _embed_call(x, Wx, bx, Wg0, degs)
    p0 = scatter_call(hs0, src_p, dst_p)
    hs1 = _mid_call(p0, hs0, dinv, bg0, g0, be0, Wg1)
    p1 = scatter_call(hs1, src_p, dst_p)
    hs2 = _mid_call(p1, hs1, dinv, bg1, g1, be1, Wg2)
    p2 = scatter_call(hs2, src_p, dst_p)
    batch2d = batch.astype(jnp.int32).reshape(1, _N)
    return _head_call(p2, hs2, dinv, bg2, g2, be2, batch2d,
                      Wf0, bf0, Wf1, bf1, Wf2, bf2)
